# Initial kernel scaffold; baseline (speedup 1.0000x reference)
import jax
import jax.numpy as jnp
from jax.experimental import pallas as pl


def kernel(x_user, weight):
    raise NotImplementedError("write your pallas kernel here")


# speedup vs baseline: 48.1934x; 48.1934x over previous
import functools

import jax
import jax.numpy as jnp
from jax import lax
from jax.experimental import pallas as pl
from jax.experimental.pallas import tpu as pltpu
from jax.experimental.pallas import tpu_sc as plsc

V = 500
D = 12
DP = 16
B = 16384
BAG = 200
NC, NS, LANES = 2, 16, 16
NW = NC * NS
BPW = B // NW
CH = 64
NCHUNK = BPW // CH
GPC = CH // LANES

_mesh = plsc.VectorSubcoreMesh(core_axis_name="c", subcore_axis_name="s")


@functools.partial(
    pl.kernel,
    out_type=jax.ShapeDtypeStruct((B * DP,), jnp.float32),
    mesh=_mesh,
    compiler_params=pltpu.CompilerParams(needs_layout_passes=False),
    scratch_types=[
        pltpu.VMEM((V * DP,), jnp.float32),
        pltpu.VMEM((CH * BAG,), jnp.int32),
        pltpu.VMEM((CH * BAG,), jnp.int32),
        pltpu.VMEM((CH * DP,), jnp.float32),
        pltpu.SemaphoreType.DMA,
        pltpu.SemaphoreType.DMA,
    ],
)
def _emb_bag(tab_hbm, idx_hbm, out_hbm, tab_v, idx_a, idx_b, out_v,
             sem_a, sem_b):
    wid = lax.axis_index("s") * NC + lax.axis_index("c")
    base_bag = wid * BPW
    pltpu.sync_copy(tab_hbm, tab_v)

    bufs = [(idx_a, sem_a), (idx_b, sem_b)]

    def start(c):
        buf, sem = bufs[c % 2]
        return pltpu.async_copy(
            idx_hbm.at[pl.ds((base_bag + c * CH) * BAG, CH * BAG)], buf, sem)

    pending = {0: start(0)}
    lane_bag = lax.iota(jnp.int32, LANES) * BAG
    lane_out = lax.iota(jnp.int32, LANES) * DP
    inv = jnp.float32(1.0 / BAG)

    for c in range(NCHUNK):
        if c + 1 < NCHUNK:
            pending[c + 1] = start(c + 1)
        pending.pop(c).wait()
        buf = bufs[c % 2][0]
        for g in range(GPC):
            addr0 = lane_bag + g * LANES * BAG

            def lbody(l, accs, buf=buf, addr0=addr0):
                rows = plsc.load_gather(buf, [addr0 + l])
                rs = rows * DP
                return tuple(accs[d] + plsc.load_gather(tab_v, [rs + d])
                             for d in range(D))

            accs = lax.fori_loop(
                0, BAG, lbody,
                tuple(jnp.zeros((LANES,), jnp.float32) for _ in range(D)))
            for d in range(D):
                plsc.store_scatter(out_v, [lane_out + (g * LANES * DP + d)],
                                   accs[d] * inv)
        pltpu.sync_copy(out_v,
                        out_hbm.at[pl.ds((base_bag + c * CH) * DP, CH * DP)])


def kernel(x_user, weight):
    xf = x_user.reshape(-1)
    wpad = jnp.pad(weight, ((0, 0), (0, DP - D))).reshape(-1)
    out = _emb_bag(wpad, xf)
    return out.reshape(B, DP)[:, :D]


# speedup vs baseline: 80.3151x; 1.6665x over previous
import functools

import jax
import jax.numpy as jnp
from jax import lax
from jax.experimental import pallas as pl
from jax.experimental.pallas import tpu as pltpu
from jax.experimental.pallas import tpu_sc as plsc

V = 500
D = 12
DP = 16
B = 16384
BAG = 200
NC, NS, LANES = 2, 16, 16
NW = NC * NS
BPW = B // NW
CH = 64
NCHUNK = BPW // CH
GPC = CH // LANES

_mesh = plsc.VectorSubcoreMesh(core_axis_name="c", subcore_axis_name="s")


@functools.partial(
    pl.kernel,
    out_type=jax.ShapeDtypeStruct((B * DP,), jnp.float32),
    mesh=_mesh,
    compiler_params=pltpu.CompilerParams(needs_layout_passes=False),
    scratch_types=[
        pltpu.VMEM((DP * 512,), jnp.float32),
        pltpu.VMEM((CH * BAG,), jnp.int32),
        pltpu.VMEM((CH * BAG,), jnp.int32),
        pltpu.VMEM((CH * DP,), jnp.float32),
        pltpu.SemaphoreType.DMA,
        pltpu.SemaphoreType.DMA,
    ],
)
def _emb_bag(tab_hbm, idx_hbm, out_hbm, tab_v, idx_a, idx_b, out_v,
             sem_a, sem_b):
    wid = lax.axis_index("s") * NC + lax.axis_index("c")
    base_bag = wid * BPW
    pltpu.sync_copy(tab_hbm, tab_v)

    bufs = [(idx_a, sem_a), (idx_b, sem_b)]

    def start(c):
        buf, sem = bufs[c % 2]
        return pltpu.async_copy(
            idx_hbm.at[pl.ds((base_bag + c * CH) * BAG, CH * BAG)], buf, sem)

    pending = {0: start(0)}
    lane_bag = lax.iota(jnp.int32, LANES) * BAG
    lane_out = lax.iota(jnp.int32, LANES) * DP
    inv = jnp.float32(1.0 / BAG)

    for c in range(NCHUNK):
        if c + 1 < NCHUNK:
            pending[c + 1] = start(c + 1)
        pending.pop(c).wait()
        buf = bufs[c % 2][0]
        for g in range(GPC):
            addr0 = lane_bag + g * LANES * BAG

            def lbody(l, accs, buf=buf, addr0=addr0):
                rows = plsc.load_gather(buf, [addr0 + l])
                return tuple(accs[d] + plsc.load_gather(tab_v, [rows + d * 512])
                             for d in range(D))

            accs = lax.fori_loop(
                0, BAG, lbody,
                tuple(jnp.zeros((LANES,), jnp.float32) for _ in range(D)))
            for d in range(D):
                plsc.store_scatter(out_v, [lane_out + (g * LANES * DP + d)],
                                   accs[d] * inv)
        pltpu.sync_copy(out_v,
                        out_hbm.at[pl.ds((base_bag + c * CH) * DP, CH * DP)])


def kernel(x_user, weight):
    xf = x_user.reshape(-1)
    wcm = jnp.pad(weight.T, ((0, DP - D), (0, 512 - V))).reshape(-1)
    out = _emb_bag(wcm, xf)
    return out.reshape(B, DP)[:, :D]


# speedup vs baseline: 96.8916x; 1.2064x over previous
import functools

import jax
import jax.numpy as jnp
from jax import lax
from jax.experimental import pallas as pl
from jax.experimental.pallas import tpu as pltpu
from jax.experimental.pallas import tpu_sc as plsc

V = 500
D = 12
DP = 16
B = 16384
BAG = 200
NC, NS, LANES = 2, 16, 16
NW = NC * NS
BPW = B // NW
CH = 64
NCHUNK = BPW // CH
GPC = CH // LANES

_mesh = plsc.VectorSubcoreMesh(core_axis_name="c", subcore_axis_name="s")


@functools.partial(
    pl.kernel,
    out_type=jax.ShapeDtypeStruct((B * DP,), jnp.float32),
    mesh=_mesh,
    compiler_params=pltpu.CompilerParams(needs_layout_passes=False),
    scratch_types=[
        pltpu.VMEM((D * 512 * LANES,), jnp.float32),
        pltpu.VMEM((CH * BAG,), jnp.int32),
        pltpu.VMEM((CH * BAG,), jnp.int32),
        pltpu.VMEM((CH * DP,), jnp.float32),
        pltpu.SemaphoreType.DMA,
        pltpu.SemaphoreType.DMA,
    ],
)
def _emb_bag(tab_hbm, idx_hbm, out_hbm, tab_v, idx_a, idx_b, out_v,
             sem_a, sem_b):
    wid = lax.axis_index("s") * NC + lax.axis_index("c")
    base_bag = wid * BPW
    pltpu.sync_copy(tab_hbm, tab_v)

    bufs = [(idx_a, sem_a), (idx_b, sem_b)]

    def start(c):
        buf, sem = bufs[c % 2]
        return pltpu.async_copy(
            idx_hbm.at[pl.ds((base_bag + c * CH) * BAG, CH * BAG)], buf, sem)

    pending = {0: start(0)}
    lane = lax.iota(jnp.int32, LANES)
    lane_bag = lane * BAG
    lane_out = lane * DP
    dim_base = [lane + d * 512 * LANES for d in range(D)]
    inv = jnp.float32(1.0 / BAG)

    for c in range(NCHUNK):
        if c + 1 < NCHUNK:
            pending[c + 1] = start(c + 1)
        pending.pop(c).wait()
        buf = bufs[c % 2][0]
        for g in range(GPC):
            addr0 = lane_bag + g * LANES * BAG

            def lbody(l, accs, buf=buf, addr0=addr0):
                rows = plsc.load_gather(buf, [addr0 + l])
                rs = rows * LANES
                return tuple(accs[d] + plsc.load_gather(tab_v, [rs + dim_base[d]])
                             for d in range(D))

            accs = lax.fori_loop(
                0, BAG, lbody,
                tuple(jnp.zeros((LANES,), jnp.float32) for _ in range(D)))
            for d in range(D):
                plsc.store_scatter(out_v, [lane_out + (g * LANES * DP + d)],
                                   accs[d] * inv)
        pltpu.sync_copy(out_v,
                        out_hbm.at[pl.ds((base_bag + c * CH) * DP, CH * DP)])


def kernel(x_user, weight):
    xf = x_user.reshape(-1)
    wrep = jnp.broadcast_to(
        jnp.pad(weight.T, ((0, 0), (0, 512 - V)))[:, :, None],
        (D, 512, LANES)).reshape(-1)
    out = _emb_bag(wrep, xf)
    return out.reshape(B, DP)[:, :D]
